# baseline (speedup 1.0000x reference)
import functools

import jax
import jax.numpy as jnp
from jax import lax
from jax.experimental import pallas as pl
from jax.experimental.pallas import tpu as pltpu
from jax.experimental.pallas import tpu_sc as plsc

NC = 2
NS = 16
NW = NC * NS
B = 16384
D = 16
BPW = B // NW
CH = 128
NCHUNK = BPW // CH

@functools.cache
def _mesh():
    return plsc.VectorSubcoreMesh(
        core_axis_name="c", subcore_axis_name="s", num_cores=NC, num_subcores=NS
    )


def _compute_groups(rela_v, link_v, pi_v, pj_v, ni_v, nj_v, tbuf, out_v):
    iota = lax.iota(jnp.int32, 16)

    def group(g, carry):
        for r in range(16):
            b = g * 16 + r
            pi = pi_v[b]
            pj = pj_v[b]
            ni = ni_v[b]
            nj = nj_v[b]
            rk = rela_v[b]
            lk = link_v[b]
            t = (nj - ni - pj + pi) * rk + (ni * nj - pi * pj) * lk
            plsc.store_scatter(tbuf, [iota, jnp.full((16,), r, jnp.int32)], t)
        acc = tbuf[0]
        for r in range(1, 16):
            acc = acc + tbuf[r]
        out_v[pl.ds(g * 16, 16)] = acc
        return carry

    lax.fori_loop(0, BPW // 16, group, 0)


def _sc_body(idx_hbm, node_hbm, rela_hbm, link_hbm, err_hbm,
             idx_v, rela_v, link_v, pi_v, pj_v, ni_v, nj_v, tbuf, out_v, sem):
    wid = lax.axis_index("s") * NC + lax.axis_index("c")
    pltpu.sync_copy(idx_hbm.at[wid], idx_v)

    copies = []
    for j in range(NCHUNK):
        sl = pl.ds(j * CH, CH)
        for col, table, dest in (
            (0, rela_hbm, rela_v), (0, link_hbm, link_v),
            (1, node_hbm, pi_v), (2, node_hbm, pj_v),
            (3, node_hbm, ni_v), (4, node_hbm, nj_v),
        ):
            copies.append(
                pltpu.async_copy(table.at[idx_v.at[col, j]], dest.at[sl], sem))
    for c in copies:
        c.wait()

    _compute_groups(rela_v, link_v, pi_v, pj_v, ni_v, nj_v, tbuf, out_v)
    pltpu.sync_copy(out_v, err_hbm.at[pl.ds(wid * BPW, BPW)])


@functools.cache
def _sc_err(interpret=False):
    return pl.kernel(
        _sc_body,
        out_type=jax.ShapeDtypeStruct((B,), jnp.float32),
        mesh=_mesh(),
        scratch_types=[
            pltpu.VMEM((5, NCHUNK, CH), jnp.int32),
            pltpu.VMEM((BPW, D), jnp.float32),
            pltpu.VMEM((BPW, D), jnp.float32),
            pltpu.VMEM((BPW, D), jnp.float32),
            pltpu.VMEM((BPW, D), jnp.float32),
            pltpu.VMEM((BPW, D), jnp.float32),
            pltpu.VMEM((BPW, D), jnp.float32),
            pltpu.VMEM((16, 16), jnp.float32),
            pltpu.VMEM((BPW,), jnp.float32),
            pltpu.SemaphoreType.DMA,
        ],
        compiler_params=pltpu.CompilerParams(
            needs_layout_passes=False, use_tc_tiling_on_sc=False),
        interpret=interpret,
    )


def _softplus_body(x_ref, o_ref):
    v = x_ref[...]
    o_ref[...] = jnp.maximum(v, 0.0) + jnp.log1p(jnp.exp(-jnp.abs(v)))


def _softplus_tc(err):
    x = err.reshape(128, 128)
    y = pl.pallas_call(
        _softplus_body,
        out_shape=jax.ShapeDtypeStruct((128, 128), jnp.float32),
    )(x)
    return y.reshape(B)


def kernel(batchVector, nodeEmbedding, relaEmbedding, linkEmbedding):
    idx = (batchVector.astype(jnp.int32)
           .reshape(NW, BPW, 5)
           .transpose(0, 2, 1)
           .reshape(NW, 5, NCHUNK, CH))
    err = _sc_err()(idx, nodeEmbedding, relaEmbedding, linkEmbedding)
    return _softplus_tc(err)
